# baseline (speedup 1.0000x reference)
import jax
import jax.numpy as jnp
from jax import lax
from jax.experimental import pallas as pl
from jax.experimental.pallas import tpu as pltpu
from jax.experimental.pallas import tpu_sc as plsc

B = 4096
D = 128
N = 16
NR = 32
NC = 2
NS = 16
NW = NC * NS
CHUNK = 128
QC = CHUNK // N
BPW = B // NW
QPW = BPW * N
L = 16
DC = D // L


def _mesh():
    return plsc.VectorSubcoreMesh(core_axis_name="c", subcore_axis_name="s")


def _wid():
    return lax.axis_index("s") * NC + lax.axis_index("c")


def _sca_body(user_idx, item_idx, adjcat, user_emb, ent_emb,
              u_out, ev0_out, ev1_out, r1_out, r2_out, e2f_out,
              idx_v, e1f_v, e2f_v, hi_v, lo_v, dstbuf, rows_v, rbuf,
              sem):
    base = _wid() * BPW
    iota = lax.iota(jnp.int32, L)

    def expand_chunk(load_ids, scatter_e):
        for g in range(CHUNK // L):
            v = load_ids(g * L + iota)
            plsc.store_scatter(hi_v, [g * L + iota], v >> 2)
            plsc.store_scatter(lo_v, [g * L + iota], (v & 3) << 5)
        pltpu.async_copy(adjcat.at[hi_v], dstbuf, sem).wait()
        for g in range(CHUNK // L):
            rows = g * L + iota
            lo = plsc.load_gather(lo_v, [rows])
            for j in range(N):
                e_j = plsc.load_gather(dstbuf, [rows, lo + j])
                r_j = plsc.load_gather(dstbuf, [rows, lo + N + j])
                scatter_e(rows, j, e_j)
                plsc.store_scatter(
                    rbuf, [rows, jnp.full((L,), j, jnp.int32)], r_j)

    pltpu.sync_copy(item_idx.at[pl.ds(base, BPW)], idx_v)
    expand_chunk(
        lambda off: plsc.load_gather(idx_v, [off]),
        lambda rows, j, e_j: plsc.store_scatter(e1f_v, [rows * N + j], e_j))
    pltpu.sync_copy(rbuf, r1_out.at[pl.ds(base, BPW)])

    pltpu.async_copy(ent_emb.at[idx_v], rows_v, sem).wait()
    pltpu.sync_copy(rows_v, ev0_out.at[pl.ds(base, BPW)])
    pltpu.sync_copy(user_idx.at[pl.ds(base, BPW)], idx_v)
    pltpu.async_copy(user_emb.at[idx_v], rows_v, sem).wait()
    pltpu.sync_copy(rows_v, u_out.at[pl.ds(base, BPW)])

    def ev1_body(c, carry):
        pltpu.async_copy(ent_emb.at[e1f_v.at[pl.ds(c * CHUNK, CHUNK)]],
                         rows_v, sem).wait()
        pltpu.sync_copy(rows_v,
                        ev1_out.at[pl.ds(base * N + c * CHUNK, CHUNK)])
        return carry

    lax.fori_loop(0, QPW // CHUNK, ev1_body, 0)

    def exp2_body(c, carry):
        expand_chunk(
            lambda off: plsc.load_gather(e1f_v, [c * CHUNK + off]),
            lambda rows, j, e_j: plsc.store_scatter(
                e2f_v, [(c * CHUNK + rows) * N + j], e_j))
        pltpu.sync_copy(rbuf, r2_out.at[pl.ds(base * N + c * CHUNK, CHUNK)])
        return carry

    lax.fori_loop(0, QPW // CHUNK, exp2_body, 0)
    pltpu.sync_copy(e2f_v, e2f_out.at[pl.ds(base * N * N, QPW * N)])


def _sc_a(user_idx, item_idx, adjcat, user_emb, ent_emb):
    return pl.kernel(
        _sca_body,
        out_type=[
            jax.ShapeDtypeStruct((B, D), jnp.float32),
            jax.ShapeDtypeStruct((B, D), jnp.float32),
            jax.ShapeDtypeStruct((B * N, D), jnp.float32),
            jax.ShapeDtypeStruct((B, N), jnp.int32),
            jax.ShapeDtypeStruct((B * N, N), jnp.int32),
            jax.ShapeDtypeStruct((B * N * N,), jnp.int32),
        ],
        mesh=_mesh(),
        compiler_params=pltpu.CompilerParams(needs_layout_passes=False),
        scratch_types=[
            pltpu.VMEM((BPW,), jnp.int32),
            pltpu.VMEM((QPW,), jnp.int32),
            pltpu.VMEM((QPW * N,), jnp.int32),
            pltpu.VMEM((CHUNK,), jnp.int32),
            pltpu.VMEM((CHUNK,), jnp.int32),
            pltpu.VMEM((CHUNK, D), jnp.int32),
            pltpu.VMEM((CHUNK, D), jnp.float32),
            pltpu.VMEM((BPW, N), jnp.int32),
            pltpu.SemaphoreType.DMA,
        ],
    )(user_idx, item_idx, adjcat, user_emb, ent_emb)


def _scb_compute(rows_v, s1c_v, aggbuf):
    iota = lax.iota(jnp.int32, L)

    def q_body(qq, carry):
        s = plsc.load_gather(s1c_v, [jnp.full((L,), qq, jnp.int32), iota])
        accs = [jnp.zeros((L,), jnp.float32) for _ in range(DC)]
        for nn in range(N):
            w_nn = jnp.broadcast_to(s[nn], (L,))
            row = jnp.full((L,), qq * N + nn, jnp.int32)
            for dc in range(DC):
                val = plsc.load_gather(rows_v, [row, dc * L + iota])
                accs[dc] = accs[dc] + w_nn * val
        for dc in range(DC):
            plsc.store_scatter(
                aggbuf, [jnp.full((L,), qq, jnp.int32), dc * L + iota],
                accs[dc])
        return carry

    lax.fori_loop(0, QC, q_body, 0)


def _scb_body(e2f, s1, ent_emb, agg1_out,
              idx_v, rows0, rows1, s1c0, s1c1, aggbuf, sem0, sem1):
    wid = _wid()
    qb = wid * QPW
    pltpu.sync_copy(e2f.at[pl.ds(wid * QPW * N, QPW * N)], idx_v)

    def issue(c, rows_v, s1c_v, sem):
        dr = pltpu.async_copy(ent_emb.at[idx_v.at[pl.ds(c * CHUNK, CHUNK)]],
                              rows_v, sem)
        ds_ = pltpu.async_copy(s1.at[pl.ds(qb + c * QC, QC)], s1c_v, sem)
        return dr, ds_

    def pair_body(cc, carry):
        c0 = 2 * cc
        dr0, ds0 = issue(c0, rows0, s1c0, sem0)
        dr1, ds1 = issue(c0 + 1, rows1, s1c1, sem1)
        dr0.wait()
        ds0.wait()
        _scb_compute(rows0, s1c0, aggbuf)
        pltpu.sync_copy(aggbuf, agg1_out.at[pl.ds(qb + c0 * QC, QC)])
        dr1.wait()
        ds1.wait()
        _scb_compute(rows1, s1c1, aggbuf)
        pltpu.sync_copy(aggbuf, agg1_out.at[pl.ds(qb + (c0 + 1) * QC, QC)])
        return carry

    lax.fori_loop(0, (QPW * N) // CHUNK // 2, pair_body, 0)


def _sc_b(e2f, s1, ent_emb):
    return pl.kernel(
        _scb_body,
        out_type=jax.ShapeDtypeStruct((B * N, D), jnp.float32),
        mesh=_mesh(),
        compiler_params=pltpu.CompilerParams(needs_layout_passes=False),
        scratch_types=[
            pltpu.VMEM((QPW * N,), jnp.int32),
            pltpu.VMEM((CHUNK, D), jnp.float32),
            pltpu.VMEM((CHUNK, D), jnp.float32),
            pltpu.VMEM((QC, N), jnp.float32),
            pltpu.VMEM((QC, N), jnp.float32),
            pltpu.VMEM((QC, D), jnp.float32),
            pltpu.SemaphoreType.DMA,
            pltpu.SemaphoreType.DMA,
        ],
    )(e2f, s1, ent_emb)


BB = 128


def _softmax(x):
    m = jnp.max(x, axis=-1, keepdims=True)
    e = jnp.exp(x - m)
    return e / jnp.sum(e, axis=-1, keepdims=True)


def _P_of(u, rel):
    return lax.dot_general(u, rel, (((1,), (1,)), ((), ())),
                           preferred_element_type=jnp.float32)


def _tcs_body(u_ref, r2_ref, rel_ref, s1_ref):
    f32 = jnp.float32
    P = _P_of(u_ref[...], rel_ref[...])
    iota_r = lax.broadcasted_iota(jnp.int32, (1, 1, NR), 2)
    r2 = r2_ref[...]
    Pexp = jnp.broadcast_to(P[:, None, :], (BB, N, NR)).reshape(BB * N, NR)
    oh2 = (r2[:, :, None] == iota_r).astype(f32)
    s1_ref[...] = _softmax(jnp.sum(oh2 * Pexp[:, None, :], axis=-1))


def _tc_scores(u, r2, rel):
    nb = B // BB
    return pl.pallas_call(
        _tcs_body,
        grid=(nb,),
        in_specs=[
            pl.BlockSpec((BB, D), lambda i: (i, 0)),
            pl.BlockSpec((BB * N, N), lambda i: (i, 0)),
            pl.BlockSpec((NR, D), lambda i: (0, 0)),
        ],
        out_specs=pl.BlockSpec((BB * N, N), lambda i: (i, 0)),
        out_shape=jax.ShapeDtypeStruct((B * N, N), jnp.float32),
    )(u, r2, rel)


def _tcf_body(u_ref, ev0_ref, ev1_ref, agg1_ref, r1_ref, rel_ref,
              w0_ref, b0_ref, w1_ref, b1_ref, out_ref):
    f32 = jnp.float32
    u = u_ref[...]
    P = _P_of(u, rel_ref[...])
    iota_r = lax.broadcasted_iota(jnp.int32, (1, 1, NR), 2)
    r1 = r1_ref[...]
    oh1 = (r1[:, :, None] == iota_r).astype(f32)
    s0 = _softmax(jnp.sum(oh1 * P[:, None, :], axis=-1))

    ev1 = ev1_ref[...].reshape(BB * N, D)
    w0 = w0_ref[...]
    b0 = b0_ref[...]
    h1 = jax.nn.relu(jnp.dot(ev1 + agg1_ref[...], w0,
                             preferred_element_type=f32) + b0)
    agg0 = jnp.sum(ev1.reshape(BB, N, D) * s0[:, :, None], axis=1)
    h0 = jax.nn.relu(jnp.dot(ev0_ref[...] + agg0, w0,
                             preferred_element_type=f32) + b0)
    agg0b = jnp.sum(h1.reshape(BB, N, D) * s0[:, :, None], axis=1)
    outv = jnp.tanh(jnp.dot(h0 + agg0b, w1_ref[...],
                            preferred_element_type=f32) + b1_ref[...])
    logits = jnp.sum(u * outv, axis=-1)
    out_ref[...] = jax.nn.sigmoid(logits)[None, None, :]


def _tc_final(u, ev0, ev1_3, agg1, r1, rel, W0, b0, W1, b1):
    nb = B // BB
    return pl.pallas_call(
        _tcf_body,
        grid=(nb,),
        in_specs=[
            pl.BlockSpec((BB, D), lambda i: (i, 0)),
            pl.BlockSpec((BB, D), lambda i: (i, 0)),
            pl.BlockSpec((BB, N, D), lambda i: (i, 0, 0)),
            pl.BlockSpec((BB * N, D), lambda i: (i, 0)),
            pl.BlockSpec((BB, N), lambda i: (i, 0)),
            pl.BlockSpec((NR, D), lambda i: (0, 0)),
            pl.BlockSpec((D, D), lambda i: (0, 0)),
            pl.BlockSpec((1, D), lambda i: (0, 0)),
            pl.BlockSpec((D, D), lambda i: (0, 0)),
            pl.BlockSpec((1, D), lambda i: (0, 0)),
        ],
        out_specs=pl.BlockSpec((1, 1, BB), lambda i: (i, 0, 0)),
        out_shape=jax.ShapeDtypeStruct((nb, 1, BB), jnp.float32),
    )(u, ev0, ev1_3, agg1, r1, rel, W0, b0, W1, b1)


def kernel(user_indices, item_indices, adj_entity, adj_relation,
           user_emb, entity_emb, relation_emb, W0, b0, W1, b1):
    adjcat = jnp.concatenate([adj_entity, adj_relation], axis=1)
    adjcat = adjcat.reshape(adj_entity.shape[0] // 4, 128)
    u, ev0, ev1, r1, r2, e2f = _sc_a(
        user_indices, item_indices, adjcat, user_emb, entity_emb)
    s1 = _tc_scores(u, r2, relation_emb)
    agg1 = _sc_b(e2f, s1, entity_emb)
    out = _tc_final(u, ev0, ev1.reshape(B, N, D), agg1, r1, relation_emb,
                    W0, b0.reshape(1, D), W1, b1.reshape(1, D))
    return out.reshape(B)


# speedup vs baseline: 1.7544x; 1.7544x over previous
import jax
import jax.numpy as jnp
from jax import lax
from jax.experimental import pallas as pl
from jax.experimental.pallas import tpu as pltpu
from jax.experimental.pallas import tpu_sc as plsc

B = 4096
D = 128
N = 16
NR = 32
NC = 2
NS = 16
NW = NC * NS
CHUNK = 128
QC = CHUNK // N
BPW = B // NW
QPW = BPW * N
L = 16
DC = D // L


def _mesh():
    return plsc.VectorSubcoreMesh(core_axis_name="c", subcore_axis_name="s")


def _wid():
    return lax.axis_index("s") * NC + lax.axis_index("c")


def _sca_body(user_idx, item_idx, adjcat, user_emb, ent_emb,
              u_out, ev0_out, ev1_out, r1_out, e2f_out, r2f_out,
              idx_v, e1f_v, e2f_v, r2f_v, hi_v, lo_v, dstbuf, rows_v, rbuf,
              sem):
    base = _wid() * BPW
    iota = lax.iota(jnp.int32, L)

    def expand_chunk(load_ids, scatter_e, scatter_r):
        for g in range(CHUNK // L):
            v = load_ids(g * L + iota)
            plsc.store_scatter(hi_v, [g * L + iota], v >> 2)
            plsc.store_scatter(lo_v, [g * L + iota], (v & 3) << 5)
        pltpu.async_copy(adjcat.at[hi_v], dstbuf, sem).wait()
        for g in range(CHUNK // L):
            rows = g * L + iota
            lo = plsc.load_gather(lo_v, [rows])
            for j in range(N):
                e_j = plsc.load_gather(dstbuf, [rows, lo + j])
                r_j = plsc.load_gather(dstbuf, [rows, lo + N + j])
                scatter_e(rows, j, e_j)
                scatter_r(rows, j, r_j)

    pltpu.sync_copy(item_idx.at[pl.ds(base, BPW)], idx_v)
    expand_chunk(
        lambda off: plsc.load_gather(idx_v, [off]),
        lambda rows, j, e_j: plsc.store_scatter(e1f_v, [rows * N + j], e_j),
        lambda rows, j, r_j: plsc.store_scatter(
            rbuf, [rows, jnp.full((L,), j, jnp.int32)], r_j))
    pltpu.sync_copy(rbuf, r1_out.at[pl.ds(base, BPW)])

    pltpu.async_copy(ent_emb.at[idx_v], rows_v, sem).wait()
    pltpu.sync_copy(rows_v, ev0_out.at[pl.ds(base, BPW)])
    pltpu.sync_copy(user_idx.at[pl.ds(base, BPW)], idx_v)
    pltpu.async_copy(user_emb.at[idx_v], rows_v, sem).wait()
    pltpu.sync_copy(rows_v, u_out.at[pl.ds(base, BPW)])

    def ev1_body(c, carry):
        pltpu.async_copy(ent_emb.at[e1f_v.at[pl.ds(c * CHUNK, CHUNK)]],
                         rows_v, sem).wait()
        pltpu.sync_copy(rows_v,
                        ev1_out.at[pl.ds(base * N + c * CHUNK, CHUNK)])
        return carry

    lax.fori_loop(0, QPW // CHUNK, ev1_body, 0)

    def exp2_body(c, carry):
        expand_chunk(
            lambda off: plsc.load_gather(e1f_v, [c * CHUNK + off]),
            lambda rows, j, e_j: plsc.store_scatter(
                e2f_v, [(c * CHUNK + rows) * N + j], e_j),
            lambda rows, j, r_j: plsc.store_scatter(
                r2f_v, [(c * CHUNK + rows) * N + j], r_j))
        return carry

    lax.fori_loop(0, QPW // CHUNK, exp2_body, 0)
    pltpu.sync_copy(e2f_v, e2f_out.at[pl.ds(base * N * N, QPW * N)])
    pltpu.sync_copy(r2f_v, r2f_out.at[pl.ds(base * N * N, QPW * N)])


def _sc_a(user_idx, item_idx, adjcat, user_emb, ent_emb):
    return pl.kernel(
        _sca_body,
        out_type=[
            jax.ShapeDtypeStruct((B, D), jnp.float32),
            jax.ShapeDtypeStruct((B, D), jnp.float32),
            jax.ShapeDtypeStruct((B * N, D), jnp.float32),
            jax.ShapeDtypeStruct((B, N), jnp.int32),
            jax.ShapeDtypeStruct((B * N * N,), jnp.int32),
            jax.ShapeDtypeStruct((B * N * N,), jnp.int32),
        ],
        mesh=_mesh(),
        compiler_params=pltpu.CompilerParams(needs_layout_passes=False),
        scratch_types=[
            pltpu.VMEM((BPW,), jnp.int32),
            pltpu.VMEM((QPW,), jnp.int32),
            pltpu.VMEM((QPW * N,), jnp.int32),
            pltpu.VMEM((QPW * N,), jnp.int32),
            pltpu.VMEM((CHUNK,), jnp.int32),
            pltpu.VMEM((CHUNK,), jnp.int32),
            pltpu.VMEM((CHUNK, D), jnp.int32),
            pltpu.VMEM((CHUNK, D), jnp.float32),
            pltpu.VMEM((BPW, N), jnp.int32),
            pltpu.SemaphoreType.DMA,
        ],
    )(user_idx, item_idx, adjcat, user_emb, ent_emb)


def _scb_compute(c, rows_v, p_v, r2f_v, aggbuf):
    iota = lax.iota(jnp.int32, L)

    def q_body(qq, carry):
        q = c * QC + qq
        r2vec = plsc.load_gather(r2f_v, [q * N + iota])
        raw = plsc.load_gather(
            p_v, [jnp.full((L,), q >> 4, jnp.int32), r2vec])
        ex = jnp.exp(raw)
        s = ex / jnp.sum(ex)
        accs = [jnp.zeros((L,), jnp.float32) for _ in range(DC)]
        for nn in range(N):
            w_nn = jnp.broadcast_to(s[nn], (L,))
            row = jnp.full((L,), qq * N + nn, jnp.int32)
            for dc in range(DC):
                val = plsc.load_gather(rows_v, [row, dc * L + iota])
                accs[dc] = accs[dc] + w_nn * val
        for dc in range(DC):
            plsc.store_scatter(
                aggbuf, [jnp.full((L,), qq, jnp.int32), dc * L + iota],
                accs[dc])
        return carry

    lax.fori_loop(0, QC, q_body, 0)


def _scb_body(e2f, r2f, p_hbm, ent_emb, agg1_out,
              idx_v, r2f_v, p_v, rows0, rows1, aggbuf, sem0, sem1):
    wid = _wid()
    qb = wid * QPW
    pltpu.sync_copy(e2f.at[pl.ds(wid * QPW * N, QPW * N)], idx_v)
    pltpu.sync_copy(r2f.at[pl.ds(wid * QPW * N, QPW * N)], r2f_v)
    pltpu.sync_copy(p_hbm.at[pl.ds(wid * BPW, BPW)], p_v)

    def issue(c, rows_v, sem):
        return pltpu.async_copy(
            ent_emb.at[idx_v.at[pl.ds(c * CHUNK, CHUNK)]], rows_v, sem)

    def pair_body(cc, carry):
        c0 = 2 * cc
        dr0 = issue(c0, rows0, sem0)
        dr1 = issue(c0 + 1, rows1, sem1)
        dr0.wait()
        _scb_compute(c0, rows0, p_v, r2f_v, aggbuf)
        pltpu.sync_copy(aggbuf, agg1_out.at[pl.ds(qb + c0 * QC, QC)])
        dr1.wait()
        _scb_compute(c0 + 1, rows1, p_v, r2f_v, aggbuf)
        pltpu.sync_copy(aggbuf, agg1_out.at[pl.ds(qb + (c0 + 1) * QC, QC)])
        return carry

    lax.fori_loop(0, (QPW * N) // CHUNK // 2, pair_body, 0)


def _sc_b(e2f, r2f, P, ent_emb):
    return pl.kernel(
        _scb_body,
        out_type=jax.ShapeDtypeStruct((B * N, D), jnp.float32),
        mesh=_mesh(),
        compiler_params=pltpu.CompilerParams(needs_layout_passes=False),
        scratch_types=[
            pltpu.VMEM((QPW * N,), jnp.int32),
            pltpu.VMEM((QPW * N,), jnp.int32),
            pltpu.VMEM((BPW, NR), jnp.float32),
            pltpu.VMEM((CHUNK, D), jnp.float32),
            pltpu.VMEM((CHUNK, D), jnp.float32),
            pltpu.VMEM((QC, D), jnp.float32),
            pltpu.SemaphoreType.DMA,
            pltpu.SemaphoreType.DMA,
        ],
    )(e2f, r2f, P, ent_emb)


def _tcp_body(u_ref, rel_ref, p_ref):
    p_ref[...] = lax.dot_general(u_ref[...], rel_ref[...],
                                 (((1,), (1,)), ((), ())),
                                 preferred_element_type=jnp.float32)


def _tc_p(u, rel):
    return pl.pallas_call(
        _tcp_body,
        grid=(1,),
        in_specs=[pl.BlockSpec((B, D), lambda i: (0, 0)),
                  pl.BlockSpec((NR, D), lambda i: (0, 0))],
        out_specs=pl.BlockSpec((B, NR), lambda i: (0, 0)),
        out_shape=jax.ShapeDtypeStruct((B, NR), jnp.float32),
    )(u, rel)


BB = 128


def _softmax(x):
    m = jnp.max(x, axis=-1, keepdims=True)
    e = jnp.exp(x - m)
    return e / jnp.sum(e, axis=-1, keepdims=True)


def _P_of(u, rel):
    return lax.dot_general(u, rel, (((1,), (1,)), ((), ())),
                           preferred_element_type=jnp.float32)


def _tcf_body(u_ref, ev0_ref, ev1_ref, agg1_ref, r1_ref, rel_ref,
              w0_ref, b0_ref, w1_ref, b1_ref, out_ref):
    f32 = jnp.float32
    u = u_ref[...]
    P = _P_of(u, rel_ref[...])
    iota_r = lax.broadcasted_iota(jnp.int32, (1, 1, NR), 2)
    r1 = r1_ref[...]
    oh1 = (r1[:, :, None] == iota_r).astype(f32)
    s0 = _softmax(jnp.sum(oh1 * P[:, None, :], axis=-1))

    ev1 = ev1_ref[...].reshape(BB * N, D)
    w0 = w0_ref[...]
    b0 = b0_ref[...]
    h1 = jax.nn.relu(jnp.dot(ev1 + agg1_ref[...], w0,
                             preferred_element_type=f32) + b0)
    agg0 = jnp.sum(ev1.reshape(BB, N, D) * s0[:, :, None], axis=1)
    h0 = jax.nn.relu(jnp.dot(ev0_ref[...] + agg0, w0,
                             preferred_element_type=f32) + b0)
    agg0b = jnp.sum(h1.reshape(BB, N, D) * s0[:, :, None], axis=1)
    outv = jnp.tanh(jnp.dot(h0 + agg0b, w1_ref[...],
                            preferred_element_type=f32) + b1_ref[...])
    logits = jnp.sum(u * outv, axis=-1)
    out_ref[...] = jax.nn.sigmoid(logits)[None, None, :]


def _tc_final(u, ev0, ev1_3, agg1, r1, rel, W0, b0, W1, b1):
    nb = B // BB
    return pl.pallas_call(
        _tcf_body,
        grid=(nb,),
        in_specs=[
            pl.BlockSpec((BB, D), lambda i: (i, 0)),
            pl.BlockSpec((BB, D), lambda i: (i, 0)),
            pl.BlockSpec((BB, N, D), lambda i: (i, 0, 0)),
            pl.BlockSpec((BB * N, D), lambda i: (i, 0)),
            pl.BlockSpec((BB, N), lambda i: (i, 0)),
            pl.BlockSpec((NR, D), lambda i: (0, 0)),
            pl.BlockSpec((D, D), lambda i: (0, 0)),
            pl.BlockSpec((1, D), lambda i: (0, 0)),
            pl.BlockSpec((D, D), lambda i: (0, 0)),
            pl.BlockSpec((1, D), lambda i: (0, 0)),
        ],
        out_specs=pl.BlockSpec((1, 1, BB), lambda i: (i, 0, 0)),
        out_shape=jax.ShapeDtypeStruct((nb, 1, BB), jnp.float32),
    )(u, ev0, ev1_3, agg1, r1, rel, W0, b0, W1, b1)


def kernel(user_indices, item_indices, adj_entity, adj_relation,
           user_emb, entity_emb, relation_emb, W0, b0, W1, b1):
    adjcat = jnp.concatenate([adj_entity, adj_relation], axis=1)
    adjcat = adjcat.reshape(adj_entity.shape[0] // 4, 128)
    u, ev0, ev1, r1, e2f, r2f = _sc_a(
        user_indices, item_indices, adjcat, user_emb, entity_emb)
    P = _tc_p(u, relation_emb)
    agg1 = _sc_b(e2f, r2f, P, entity_emb)
    out = _tc_final(u, ev0, ev1.reshape(B, N, D), agg1, r1, relation_emb,
                    W0, b0.reshape(1, D), W1, b1.reshape(1, D))
    return out.reshape(B)


# speedup vs baseline: 1.8060x; 1.0294x over previous
import jax
import jax.numpy as jnp
from jax import lax
from jax.experimental import pallas as pl
from jax.experimental.pallas import tpu as pltpu
from jax.experimental.pallas import tpu_sc as plsc

B = 4096
D = 128
N = 16
NR = 32
NC = 2
NS = 16
NW = NC * NS
CHUNK = 128
QC = CHUNK // N
BPW = B // NW
QPW = BPW * N
L = 16
DC = D // L


def _mesh():
    return plsc.VectorSubcoreMesh(core_axis_name="c", subcore_axis_name="s")


def _wid():
    return lax.axis_index("s") * NC + lax.axis_index("c")


def _sca_body(user_idx, item_idx, adjcat, user_emb, ent_emb,
              u_out, ev0_out, ev1_out, r1_out, e2f_out, r2f_out,
              idx_v, e1f_v, e2f_v, r2f_v, hi_v, lo_v, dstbuf, rows_v, rbuf,
              sem):
    base = _wid() * BPW
    iota = lax.iota(jnp.int32, L)

    def expand_chunk(load_ids, scatter_e, scatter_r):
        for g in range(CHUNK // L):
            v = load_ids(g * L + iota)
            plsc.store_scatter(hi_v, [g * L + iota], v >> 2)
            plsc.store_scatter(lo_v, [g * L + iota], (v & 3) << 5)
        pltpu.async_copy(adjcat.at[hi_v], dstbuf, sem).wait()
        for g in range(CHUNK // L):
            rows = g * L + iota
            lo = plsc.load_gather(lo_v, [rows])
            for j in range(N):
                e_j = plsc.load_gather(dstbuf, [rows, lo + j])
                r_j = plsc.load_gather(dstbuf, [rows, lo + N + j])
                scatter_e(rows, j, e_j)
                scatter_r(rows, j, r_j)

    pltpu.sync_copy(item_idx.at[pl.ds(base, BPW)], idx_v)
    expand_chunk(
        lambda off: plsc.load_gather(idx_v, [off]),
        lambda rows, j, e_j: plsc.store_scatter(e1f_v, [rows * N + j], e_j),
        lambda rows, j, r_j: plsc.store_scatter(
            rbuf, [rows, jnp.full((L,), j, jnp.int32)], r_j))
    pltpu.sync_copy(rbuf, r1_out.at[pl.ds(base, BPW)])

    pltpu.async_copy(ent_emb.at[idx_v], rows_v, sem).wait()
    pltpu.sync_copy(rows_v, ev0_out.at[pl.ds(base, BPW)])
    pltpu.sync_copy(user_idx.at[pl.ds(base, BPW)], idx_v)
    pltpu.async_copy(user_emb.at[idx_v], rows_v, sem).wait()
    pltpu.sync_copy(rows_v, u_out.at[pl.ds(base, BPW)])

    def ev1_body(c, carry):
        pltpu.async_copy(ent_emb.at[e1f_v.at[pl.ds(c * CHUNK, CHUNK)]],
                         rows_v, sem).wait()
        pltpu.sync_copy(rows_v,
                        ev1_out.at[pl.ds(base * N + c * CHUNK, CHUNK)])
        return carry

    lax.fori_loop(0, QPW // CHUNK, ev1_body, 0)

    def exp2_body(c, carry):
        expand_chunk(
            lambda off: plsc.load_gather(e1f_v, [c * CHUNK + off]),
            lambda rows, j, e_j: plsc.store_scatter(
                e2f_v, [(c * CHUNK + rows) * N + j], e_j),
            lambda rows, j, r_j: plsc.store_scatter(
                r2f_v, [(c * CHUNK + rows) * N + j], r_j))
        return carry

    lax.fori_loop(0, QPW // CHUNK, exp2_body, 0)
    pltpu.sync_copy(e2f_v, e2f_out.at[pl.ds(base * N * N, QPW * N)])
    pltpu.sync_copy(r2f_v, r2f_out.at[pl.ds(base * N * N, QPW * N)])


def _sc_a(user_idx, item_idx, adjcat, user_emb, ent_emb):
    return pl.kernel(
        _sca_body,
        out_type=[
            jax.ShapeDtypeStruct((B, D), jnp.float32),
            jax.ShapeDtypeStruct((B, D), jnp.float32),
            jax.ShapeDtypeStruct((B * N, D), jnp.float32),
            jax.ShapeDtypeStruct((B, N), jnp.int32),
            jax.ShapeDtypeStruct((B * N * N,), jnp.int32),
            jax.ShapeDtypeStruct((B * N * N,), jnp.int32),
        ],
        mesh=_mesh(),
        compiler_params=pltpu.CompilerParams(needs_layout_passes=False),
        scratch_types=[
            pltpu.VMEM((BPW,), jnp.int32),
            pltpu.VMEM((QPW,), jnp.int32),
            pltpu.VMEM((QPW * N,), jnp.int32),
            pltpu.VMEM((QPW * N,), jnp.int32),
            pltpu.VMEM((CHUNK,), jnp.int32),
            pltpu.VMEM((CHUNK,), jnp.int32),
            pltpu.VMEM((CHUNK, D), jnp.int32),
            pltpu.VMEM((CHUNK, D), jnp.float32),
            pltpu.VMEM((BPW, N), jnp.int32),
            pltpu.SemaphoreType.DMA,
        ],
    )(user_idx, item_idx, adjcat, user_emb, ent_emb)


NBUF = 4


def _scb_compute(c, rows_v, p_v, r2c_v, aggbuf):
    iota = lax.iota(jnp.int32, L)

    def q_body(qq, carry):
        q = c * QC + qq
        r2vec = plsc.load_gather(r2c_v, [qq * N + iota])
        raw = plsc.load_gather(
            p_v, [jnp.full((L,), q >> 4, jnp.int32), r2vec])
        ex = jnp.exp(raw)
        s = ex / jnp.sum(ex)
        accs = [jnp.zeros((L,), jnp.float32) for _ in range(DC)]
        for nn in range(N):
            w_nn = jnp.broadcast_to(s[nn], (L,))
            row = jnp.full((L,), qq * N + nn, jnp.int32)
            for dc in range(DC):
                val = plsc.load_gather(rows_v, [row, dc * L + iota])
                accs[dc] = accs[dc] + w_nn * val
        for dc in range(DC):
            plsc.store_scatter(
                aggbuf, [jnp.full((L,), qq, jnp.int32), dc * L + iota],
                accs[dc])
        return carry

    lax.fori_loop(0, QC, q_body, 0)


def _scb_body(e2f, r2f, p_hbm, ent_emb, agg1_out,
              idx_v, p_v, rowsb, r2cb, aggbuf, *sems):
    wid = _wid()
    qb = wid * QPW
    fb = wid * QPW * N
    pltpu.sync_copy(e2f.at[pl.ds(fb, QPW * N)], idx_v)
    pltpu.sync_copy(p_hbm.at[pl.ds(wid * BPW, BPW)], p_v)

    rows = rowsb
    r2c = r2cb

    def issue(c, k):
        dr = pltpu.async_copy(
            ent_emb.at[idx_v.at[pl.ds(c * CHUNK, CHUNK)]], rows[k], sems[k])
        d2 = pltpu.async_copy(r2f.at[pl.ds(fb + c * CHUNK, CHUNK)],
                              r2c[k], sems[k])
        return dr, d2

    def ring_body(cc, carry):
        c0 = NBUF * cc
        descs = [issue(c0 + k, k) for k in range(NBUF)]
        for k in range(NBUF):
            dr, d2 = descs[k]
            dr.wait()
            d2.wait()
            _scb_compute(c0 + k, rows[k], p_v, r2c[k], aggbuf)
            pltpu.sync_copy(aggbuf,
                            agg1_out.at[pl.ds(qb + (c0 + k) * QC, QC)])
        return carry

    lax.fori_loop(0, (QPW * N) // CHUNK // NBUF, ring_body, 0)


def _sc_b(e2f, r2f, P, ent_emb):
    def body(e2f_, r2f_, p_, ent_, out_, idx_v, p_v, *rest):
        rowsb = rest[:NBUF]
        r2cb = rest[NBUF:2 * NBUF]
        aggbuf = rest[2 * NBUF]
        sems = rest[2 * NBUF + 1:]
        _scb_body(e2f_, r2f_, p_, ent_, out_, idx_v, p_v, rowsb, r2cb,
                  aggbuf, *sems)

    return pl.kernel(
        body,
        out_type=jax.ShapeDtypeStruct((B * N, D), jnp.float32),
        mesh=_mesh(),
        compiler_params=pltpu.CompilerParams(needs_layout_passes=False),
        scratch_types=(
            [pltpu.VMEM((QPW * N,), jnp.int32),
             pltpu.VMEM((BPW, NR), jnp.float32)]
            + [pltpu.VMEM((CHUNK, D), jnp.float32) for _ in range(NBUF)]
            + [pltpu.VMEM((CHUNK,), jnp.int32) for _ in range(NBUF)]
            + [pltpu.VMEM((QC, D), jnp.float32)]
            + [pltpu.SemaphoreType.DMA for _ in range(NBUF)]
        ),
    )(e2f, r2f, P, ent_emb)


def _tcp_body(u_ref, rel_ref, p_ref):
    p_ref[...] = lax.dot_general(u_ref[...], rel_ref[...],
                                 (((1,), (1,)), ((), ())),
                                 preferred_element_type=jnp.float32)


def _tc_p(u, rel):
    return pl.pallas_call(
        _tcp_body,
        grid=(1,),
        in_specs=[pl.BlockSpec((B, D), lambda i: (0, 0)),
                  pl.BlockSpec((NR, D), lambda i: (0, 0))],
        out_specs=pl.BlockSpec((B, NR), lambda i: (0, 0)),
        out_shape=jax.ShapeDtypeStruct((B, NR), jnp.float32),
    )(u, rel)


BB = 128


def _softmax(x):
    m = jnp.max(x, axis=-1, keepdims=True)
    e = jnp.exp(x - m)
    return e / jnp.sum(e, axis=-1, keepdims=True)


def _P_of(u, rel):
    return lax.dot_general(u, rel, (((1,), (1,)), ((), ())),
                           preferred_element_type=jnp.float32)


def _tcf_body(u_ref, ev0_ref, ev1_ref, agg1_ref, r1_ref, rel_ref,
              w0_ref, b0_ref, w1_ref, b1_ref, out_ref):
    f32 = jnp.float32
    u = u_ref[...]
    P = _P_of(u, rel_ref[...])
    iota_r = lax.broadcasted_iota(jnp.int32, (1, 1, NR), 2)
    r1 = r1_ref[...]
    oh1 = (r1[:, :, None] == iota_r).astype(f32)
    s0 = _softmax(jnp.sum(oh1 * P[:, None, :], axis=-1))

    ev1 = ev1_ref[...].reshape(BB * N, D)
    w0 = w0_ref[...]
    b0 = b0_ref[...]
    h1 = jax.nn.relu(jnp.dot(ev1 + agg1_ref[...], w0,
                             preferred_element_type=f32) + b0)
    agg0 = jnp.sum(ev1.reshape(BB, N, D) * s0[:, :, None], axis=1)
    h0 = jax.nn.relu(jnp.dot(ev0_ref[...] + agg0, w0,
                             preferred_element_type=f32) + b0)
    agg0b = jnp.sum(h1.reshape(BB, N, D) * s0[:, :, None], axis=1)
    outv = jnp.tanh(jnp.dot(h0 + agg0b, w1_ref[...],
                            preferred_element_type=f32) + b1_ref[...])
    logits = jnp.sum(u * outv, axis=-1)
    out_ref[...] = jax.nn.sigmoid(logits)[None, None, :]


def _tc_final(u, ev0, ev1_3, agg1, r1, rel, W0, b0, W1, b1):
    nb = B // BB
    return pl.pallas_call(
        _tcf_body,
        grid=(nb,),
        in_specs=[
            pl.BlockSpec((BB, D), lambda i: (i, 0)),
            pl.BlockSpec((BB, D), lambda i: (i, 0)),
            pl.BlockSpec((BB, N, D), lambda i: (i, 0, 0)),
            pl.BlockSpec((BB * N, D), lambda i: (i, 0)),
            pl.BlockSpec((BB, N), lambda i: (i, 0)),
            pl.BlockSpec((NR, D), lambda i: (0, 0)),
            pl.BlockSpec((D, D), lambda i: (0, 0)),
            pl.BlockSpec((1, D), lambda i: (0, 0)),
            pl.BlockSpec((D, D), lambda i: (0, 0)),
            pl.BlockSpec((1, D), lambda i: (0, 0)),
        ],
        out_specs=pl.BlockSpec((1, 1, BB), lambda i: (i, 0, 0)),
        out_shape=jax.ShapeDtypeStruct((nb, 1, BB), jnp.float32),
    )(u, ev0, ev1_3, agg1, r1, rel, W0, b0, W1, b1)


def kernel(user_indices, item_indices, adj_entity, adj_relation,
           user_emb, entity_emb, relation_emb, W0, b0, W1, b1):
    adjcat = jnp.concatenate([adj_entity, adj_relation], axis=1)
    adjcat = adjcat.reshape(adj_entity.shape[0] // 4, 128)
    u, ev0, ev1, r1, e2f, r2f = _sc_a(
        user_indices, item_indices, adjcat, user_emb, entity_emb)
    P = _tc_p(u, relation_emb)
    agg1 = _sc_b(e2f, r2f, P, entity_emb)
    out = _tc_final(u, ev0, ev1.reshape(B, N, D), agg1, r1, relation_emb,
                    W0, b0.reshape(1, D), W1, b1.reshape(1, D))
    return out.reshape(B)


# speedup vs baseline: 1.8334x; 1.0152x over previous
import jax
import jax.numpy as jnp
from jax import lax
from jax.experimental import pallas as pl
from jax.experimental.pallas import tpu as pltpu
from jax.experimental.pallas import tpu_sc as plsc

B = 4096
D = 128
N = 16
NR = 32
NC = 2
NS = 16
NW = NC * NS
CHUNK = 128
QC = CHUNK // N
BPW = B // NW
QPW = BPW * N
L = 16
DC = D // L


def _mesh():
    return plsc.VectorSubcoreMesh(core_axis_name="c", subcore_axis_name="s")


def _wid():
    return lax.axis_index("s") * NC + lax.axis_index("c")


def _sca_body(user_idx, item_idx, adjcat, user_emb, ent_emb,
              u_out, ev0_out, ev1_out, r1_out, e2f_out, r2f_out,
              idx_v, e1f_v, e2f_v, r2f_v, hi_v, lo_v, dstbuf, rows_v, rbuf,
              sem):
    base = _wid() * BPW
    iota = lax.iota(jnp.int32, L)

    def expand_chunk(load_ids, scatter_e, scatter_r):
        for g in range(CHUNK // L):
            v = load_ids(g * L + iota)
            plsc.store_scatter(hi_v, [g * L + iota], v >> 2)
            plsc.store_scatter(lo_v, [g * L + iota], (v & 3) << 5)
        pltpu.async_copy(adjcat.at[hi_v], dstbuf, sem).wait()
        for g in range(CHUNK // L):
            rows = g * L + iota
            lo = plsc.load_gather(lo_v, [rows])
            for j in range(N):
                e_j = plsc.load_gather(dstbuf, [rows, lo + j])
                r_j = plsc.load_gather(dstbuf, [rows, lo + N + j])
                scatter_e(rows, j, e_j)
                scatter_r(rows, j, r_j)

    pltpu.sync_copy(item_idx.at[pl.ds(base, BPW)], idx_v)
    expand_chunk(
        lambda off: plsc.load_gather(idx_v, [off]),
        lambda rows, j, e_j: plsc.store_scatter(e1f_v, [rows * N + j], e_j),
        lambda rows, j, r_j: plsc.store_scatter(
            rbuf, [rows, jnp.full((L,), j, jnp.int32)], r_j))
    pltpu.sync_copy(rbuf, r1_out.at[pl.ds(base, BPW)])

    pltpu.async_copy(ent_emb.at[idx_v], rows_v, sem).wait()
    pltpu.sync_copy(rows_v, ev0_out.at[pl.ds(base, BPW)])
    pltpu.sync_copy(user_idx.at[pl.ds(base, BPW)], idx_v)
    pltpu.async_copy(user_emb.at[idx_v], rows_v, sem).wait()
    pltpu.sync_copy(rows_v, u_out.at[pl.ds(base, BPW)])

    def ev1_body(c, carry):
        pltpu.async_copy(ent_emb.at[e1f_v.at[pl.ds(c * CHUNK, CHUNK)]],
                         rows_v, sem).wait()
        pltpu.sync_copy(rows_v,
                        ev1_out.at[pl.ds(base * N + c * CHUNK, CHUNK)])
        return carry

    lax.fori_loop(0, QPW // CHUNK, ev1_body, 0)

    def exp2_body(c, carry):
        expand_chunk(
            lambda off: plsc.load_gather(e1f_v, [c * CHUNK + off]),
            lambda rows, j, e_j: plsc.store_scatter(
                e2f_v, [(c * CHUNK + rows) * N + j], e_j),
            lambda rows, j, r_j: plsc.store_scatter(
                r2f_v, [(c * CHUNK + rows) * N + j], r_j))
        return carry

    lax.fori_loop(0, QPW // CHUNK, exp2_body, 0)
    pltpu.sync_copy(e2f_v, e2f_out.at[pl.ds(base * N * N, QPW * N)])
    pltpu.sync_copy(r2f_v, r2f_out.at[pl.ds(base * N * N, QPW * N)])


def _sc_a(user_idx, item_idx, adjcat, user_emb, ent_emb):
    return pl.kernel(
        _sca_body,
        out_type=[
            jax.ShapeDtypeStruct((B, D), jnp.float32),
            jax.ShapeDtypeStruct((B, D), jnp.float32),
            jax.ShapeDtypeStruct((B * N, D), jnp.float32),
            jax.ShapeDtypeStruct((B, N), jnp.int32),
            jax.ShapeDtypeStruct((B * N * N,), jnp.int32),
            jax.ShapeDtypeStruct((B * N * N,), jnp.int32),
        ],
        mesh=_mesh(),
        compiler_params=pltpu.CompilerParams(needs_layout_passes=False),
        scratch_types=[
            pltpu.VMEM((BPW,), jnp.int32),
            pltpu.VMEM((QPW,), jnp.int32),
            pltpu.VMEM((QPW * N,), jnp.int32),
            pltpu.VMEM((QPW * N,), jnp.int32),
            pltpu.VMEM((CHUNK,), jnp.int32),
            pltpu.VMEM((CHUNK,), jnp.int32),
            pltpu.VMEM((CHUNK, D), jnp.int32),
            pltpu.VMEM((CHUNK, D), jnp.float32),
            pltpu.VMEM((BPW, N), jnp.int32),
            pltpu.SemaphoreType.DMA,
        ],
    )(user_idx, item_idx, adjcat, user_emb, ent_emb)


NBUF = 4


def _scb_compute(c, rows_v, p_v, r2c_v, aggbuf):
    iota = lax.iota(jnp.int32, L)

    def q_body(qq, carry):
        q = c * QC + qq
        r2vec = plsc.load_gather(r2c_v, [qq * N + iota])
        raw = plsc.load_gather(
            p_v, [jnp.full((L,), q >> 4, jnp.int32), r2vec])
        ex = jnp.exp(raw)
        s = ex / jnp.sum(ex)
        accs = [jnp.zeros((L,), jnp.float32) for _ in range(DC)]
        for nn in range(N):
            w_nn = jnp.broadcast_to(s[nn], (L,))
            row = jnp.full((L,), qq * N + nn, jnp.int32)
            for dc in range(DC):
                val = plsc.load_gather(rows_v, [row, dc * L + iota])
                accs[dc] = accs[dc] + w_nn * val
        for dc in range(DC):
            plsc.store_scatter(
                aggbuf, [jnp.full((L,), qq, jnp.int32), dc * L + iota],
                accs[dc])
        return carry

    lax.fori_loop(0, QC, q_body, 0)


def _scb_body(e2f, r2f, p_hbm, ent_emb, agg1_out,
              idx_v, p_v, rowsb, r2cb, aggbuf, *sems):
    wid = _wid()
    qb = wid * QPW
    fb = wid * QPW * N
    pltpu.sync_copy(e2f.at[pl.ds(fb, QPW * N)], idx_v)
    pltpu.sync_copy(p_hbm.at[pl.ds(wid * BPW, BPW)], p_v)

    rows = rowsb
    r2c = r2cb

    def issue(c, k):
        dr = pltpu.async_copy(
            ent_emb.at[idx_v.at[pl.ds(c * CHUNK, CHUNK)]], rows[k], sems[k])
        d2 = pltpu.async_copy(r2f.at[pl.ds(fb + c * CHUNK, CHUNK)],
                              r2c[k], sems[k])
        return dr, d2

    def ring_body(cc, carry):
        c0 = NBUF * cc
        descs = [issue(c0 + k, k) for k in range(NBUF)]
        for k in range(NBUF):
            dr, d2 = descs[k]
            dr.wait()
            d2.wait()
            _scb_compute(c0 + k, rows[k], p_v, r2c[k], aggbuf)
            pltpu.sync_copy(aggbuf,
                            agg1_out.at[pl.ds(qb + (c0 + k) * QC, QC)])
        return carry

    lax.fori_loop(0, (QPW * N) // CHUNK // NBUF, ring_body, 0)


def _sc_b(e2f, r2f, P, ent_emb):
    def body(e2f_, r2f_, p_, ent_, out_, idx_v, p_v, *rest):
        rowsb = rest[:NBUF]
        r2cb = rest[NBUF:2 * NBUF]
        aggbuf = rest[2 * NBUF]
        sems = rest[2 * NBUF + 1:]
        _scb_body(e2f_, r2f_, p_, ent_, out_, idx_v, p_v, rowsb, r2cb,
                  aggbuf, *sems)

    return pl.kernel(
        body,
        out_type=jax.ShapeDtypeStruct((B * N, D), jnp.float32),
        mesh=_mesh(),
        compiler_params=pltpu.CompilerParams(needs_layout_passes=False),
        scratch_types=(
            [pltpu.VMEM((QPW * N,), jnp.int32),
             pltpu.VMEM((BPW, NR), jnp.float32)]
            + [pltpu.VMEM((CHUNK, D), jnp.float32) for _ in range(NBUF)]
            + [pltpu.VMEM((CHUNK,), jnp.int32) for _ in range(NBUF)]
            + [pltpu.VMEM((QC, D), jnp.float32)]
            + [pltpu.SemaphoreType.DMA for _ in range(NBUF)]
        ),
    )(e2f, r2f, P, ent_emb)


def _tcp_body(u_ref, rel_ref, p_ref):
    p_ref[...] = lax.dot_general(u_ref[...], rel_ref[...],
                                 (((1,), (1,)), ((), ())),
                                 preferred_element_type=jnp.float32)


def _tc_p(u, rel):
    return pl.pallas_call(
        _tcp_body,
        grid=(1,),
        in_specs=[pl.BlockSpec((B, D), lambda i: (0, 0)),
                  pl.BlockSpec((NR, D), lambda i: (0, 0))],
        out_specs=pl.BlockSpec((B, NR), lambda i: (0, 0)),
        out_shape=jax.ShapeDtypeStruct((B, NR), jnp.float32),
    )(u, rel)


BB = 512


def _softmax(x):
    m = jnp.max(x, axis=-1, keepdims=True)
    e = jnp.exp(x - m)
    return e / jnp.sum(e, axis=-1, keepdims=True)


def _P_of(u, rel):
    return lax.dot_general(u, rel, (((1,), (1,)), ((), ())),
                           preferred_element_type=jnp.float32)


def _tcf_body(u_ref, ev0_ref, ev1_ref, agg1_ref, r1_ref, rel_ref,
              w0_ref, b0_ref, w1_ref, b1_ref, out_ref):
    f32 = jnp.float32
    u = u_ref[...]
    P = _P_of(u, rel_ref[...])
    iota_r = lax.broadcasted_iota(jnp.int32, (1, 1, NR), 2)
    r1 = r1_ref[...]
    oh1 = (r1[:, :, None] == iota_r).astype(f32)
    s0 = _softmax(jnp.sum(oh1 * P[:, None, :], axis=-1))

    ev1 = ev1_ref[...].reshape(BB * N, D)
    w0 = w0_ref[...]
    b0 = b0_ref[...]
    h1 = jax.nn.relu(jnp.dot(ev1 + agg1_ref[...], w0,
                             preferred_element_type=f32) + b0)
    agg0 = jnp.sum(ev1.reshape(BB, N, D) * s0[:, :, None], axis=1)
    h0 = jax.nn.relu(jnp.dot(ev0_ref[...] + agg0, w0,
                             preferred_element_type=f32) + b0)
    agg0b = jnp.sum(h1.reshape(BB, N, D) * s0[:, :, None], axis=1)
    outv = jnp.tanh(jnp.dot(h0 + agg0b, w1_ref[...],
                            preferred_element_type=f32) + b1_ref[...])
    logits = jnp.sum(u * outv, axis=-1)
    out_ref[...] = jax.nn.sigmoid(logits)[None, None, :]


def _tc_final(u, ev0, ev1_3, agg1, r1, rel, W0, b0, W1, b1):
    nb = B // BB
    return pl.pallas_call(
        _tcf_body,
        grid=(nb,),
        in_specs=[
            pl.BlockSpec((BB, D), lambda i: (i, 0)),
            pl.BlockSpec((BB, D), lambda i: (i, 0)),
            pl.BlockSpec((BB, N, D), lambda i: (i, 0, 0)),
            pl.BlockSpec((BB * N, D), lambda i: (i, 0)),
            pl.BlockSpec((BB, N), lambda i: (i, 0)),
            pl.BlockSpec((NR, D), lambda i: (0, 0)),
            pl.BlockSpec((D, D), lambda i: (0, 0)),
            pl.BlockSpec((1, D), lambda i: (0, 0)),
            pl.BlockSpec((D, D), lambda i: (0, 0)),
            pl.BlockSpec((1, D), lambda i: (0, 0)),
        ],
        out_specs=pl.BlockSpec((1, 1, BB), lambda i: (i, 0, 0)),
        out_shape=jax.ShapeDtypeStruct((nb, 1, BB), jnp.float32),
    )(u, ev0, ev1_3, agg1, r1, rel, W0, b0, W1, b1)


def kernel(user_indices, item_indices, adj_entity, adj_relation,
           user_emb, entity_emb, relation_emb, W0, b0, W1, b1):
    adjcat = jnp.concatenate([adj_entity, adj_relation], axis=1)
    adjcat = adjcat.reshape(adj_entity.shape[0] // 4, 128)
    u, ev0, ev1, r1, e2f, r2f = _sc_a(
        user_indices, item_indices, adjcat, user_emb, entity_emb)
    P = _tc_p(u, relation_emb)
    agg1 = _sc_b(e2f, r2f, P, entity_emb)
    out = _tc_final(u, ev0, ev1.reshape(B, N, D), agg1, r1, relation_emb,
                    W0, b0.reshape(1, D), W1, b1.reshape(1, D))
    return out.reshape(B)


# speedup vs baseline: 1.8824x; 1.0267x over previous
import jax
import jax.numpy as jnp
from jax import lax
from jax.experimental import pallas as pl
from jax.experimental.pallas import tpu as pltpu
from jax.experimental.pallas import tpu_sc as plsc

B = 4096
D = 128
N = 16
NR = 32
NC = 2
NS = 16
NW = NC * NS
CHUNK = 128
QC = CHUNK // N
BPW = B // NW
QPW = BPW * N
L = 16
DC = D // L


def _mesh():
    return plsc.VectorSubcoreMesh(core_axis_name="c", subcore_axis_name="s")


def _wid():
    return lax.axis_index("s") * NC + lax.axis_index("c")


def _sca_body(user_idx, item_idx, adjcat, user_emb, ent_emb,
              u_out, ev0_out, ev1_out, r1_out, e2f_out, r2f_out,
              idx_v, e1f_v, e2c_v, r2c_v, hi_v, lo_v, dstbuf, rows_v, rbuf,
              hi_b, lo_b, dst_b, rows_b, sem, sem_b):
    base = _wid() * BPW
    iota = lax.iota(jnp.int32, L)
    hi2, lo2 = (hi_v, hi_b), (lo_v, lo_b)
    dst2, rows2 = (dstbuf, dst_b), (rows_v, rows_b)
    sem2 = (sem, sem_b)

    def expand_chunk(load_ids, scatter_e, scatter_r):
        for g in range(CHUNK // L):
            v = load_ids(g * L + iota)
            plsc.store_scatter(hi_v, [g * L + iota], v >> 2)
            plsc.store_scatter(lo_v, [g * L + iota], (v & 3) << 5)
        pltpu.async_copy(adjcat.at[hi_v], dstbuf, sem).wait()
        for g in range(CHUNK // L):
            rows = g * L + iota
            lo = plsc.load_gather(lo_v, [rows])
            for j in range(N):
                e_j = plsc.load_gather(dstbuf, [rows, lo + j])
                r_j = plsc.load_gather(dstbuf, [rows, lo + N + j])
                scatter_e(rows, j, e_j)
                scatter_r(rows, j, r_j)

    pltpu.sync_copy(item_idx.at[pl.ds(base, BPW)], idx_v)
    expand_chunk(
        lambda off: plsc.load_gather(idx_v, [off]),
        lambda rows, j, e_j: plsc.store_scatter(e1f_v, [rows * N + j], e_j),
        lambda rows, j, r_j: plsc.store_scatter(
            rbuf, [rows, jnp.full((L,), j, jnp.int32)], r_j))
    pltpu.sync_copy(rbuf, r1_out.at[pl.ds(base, BPW)])

    pltpu.async_copy(ent_emb.at[idx_v], rows_v, sem).wait()
    pltpu.sync_copy(rows_v, ev0_out.at[pl.ds(base, BPW)])
    pltpu.sync_copy(user_idx.at[pl.ds(base, BPW)], idx_v)
    pltpu.async_copy(user_emb.at[idx_v], rows_v, sem).wait()
    pltpu.sync_copy(rows_v, u_out.at[pl.ds(base, BPW)])

    def prep(c, hi_b, lo_b):
        for g in range(CHUNK // L):
            v = plsc.load_gather(e1f_v, [c * CHUNK + g * L + iota])
            plsc.store_scatter(hi_b, [g * L + iota], v >> 2)
            plsc.store_scatter(lo_b, [g * L + iota], (v & 3) << 5)

    def issue_cd(c, k):
        prep(c, hi2[k], lo2[k])
        da = pltpu.async_copy(adjcat.at[hi2[k]], dst2[k], sem2[k])
        de = pltpu.async_copy(ent_emb.at[e1f_v.at[pl.ds(c * CHUNK, CHUNK)]],
                              rows2[k], sem2[k])
        return da, de

    def drain_cd(c, k, da, de):
        da.wait()
        for g in range(CHUNK // L):
            rows = g * L + iota
            lo = plsc.load_gather(lo2[k], [rows])
            for j in range(N):
                e_j = plsc.load_gather(dst2[k], [rows, lo + j])
                r_j = plsc.load_gather(dst2[k], [rows, lo + N + j])
                plsc.store_scatter(e2c_v, [rows * N + j], e_j)
                plsc.store_scatter(r2c_v, [rows * N + j], r_j)
        fb = base * N * N + c * CHUNK * N
        pltpu.sync_copy(e2c_v, e2f_out.at[pl.ds(fb, CHUNK * N)])
        pltpu.sync_copy(r2c_v, r2f_out.at[pl.ds(fb, CHUNK * N)])
        de.wait()
        pltpu.sync_copy(rows2[k],
                        ev1_out.at[pl.ds(base * N + c * CHUNK, CHUNK)])

    def cd_body(cc, carry):
        c0 = 2 * cc
        da0, de0 = issue_cd(c0, 0)
        da1, de1 = issue_cd(c0 + 1, 1)
        drain_cd(c0, 0, da0, de0)
        drain_cd(c0 + 1, 1, da1, de1)
        return carry

    lax.fori_loop(0, QPW // CHUNK // 2, cd_body, 0)


def _sc_a(user_idx, item_idx, adjcat, user_emb, ent_emb):
    return pl.kernel(
        _sca_body,
        out_type=[
            jax.ShapeDtypeStruct((B, D), jnp.float32),
            jax.ShapeDtypeStruct((B, D), jnp.float32),
            jax.ShapeDtypeStruct((B * N, D), jnp.float32),
            jax.ShapeDtypeStruct((B, N), jnp.int32),
            jax.ShapeDtypeStruct((B * N * N,), jnp.int32),
            jax.ShapeDtypeStruct((B * N * N,), jnp.int32),
        ],
        mesh=_mesh(),
        compiler_params=pltpu.CompilerParams(needs_layout_passes=False),
        scratch_types=[
            pltpu.VMEM((BPW,), jnp.int32),
            pltpu.VMEM((QPW,), jnp.int32),
            pltpu.VMEM((CHUNK * N,), jnp.int32),
            pltpu.VMEM((CHUNK * N,), jnp.int32),
            pltpu.VMEM((CHUNK,), jnp.int32),
            pltpu.VMEM((CHUNK,), jnp.int32),
            pltpu.VMEM((CHUNK, D), jnp.int32),
            pltpu.VMEM((CHUNK, D), jnp.float32),
            pltpu.VMEM((BPW, N), jnp.int32),
            pltpu.VMEM((CHUNK,), jnp.int32),
            pltpu.VMEM((CHUNK,), jnp.int32),
            pltpu.VMEM((CHUNK, D), jnp.int32),
            pltpu.VMEM((CHUNK, D), jnp.float32),
            pltpu.SemaphoreType.DMA,
            pltpu.SemaphoreType.DMA,
        ],
    )(user_idx, item_idx, adjcat, user_emb, ent_emb)


NBUF = 4


def _scb_compute(c, rows_v, p_v, r2c_v, aggbuf):
    iota = lax.iota(jnp.int32, L)

    def q_body(qq, carry):
        q = c * QC + qq
        r2vec = plsc.load_gather(r2c_v, [qq * N + iota])
        raw = plsc.load_gather(
            p_v, [jnp.full((L,), q >> 4, jnp.int32), r2vec])
        ex = jnp.exp(raw)
        s = ex / jnp.sum(ex)
        accs = [jnp.zeros((L,), jnp.float32) for _ in range(DC)]
        for nn in range(N):
            w_nn = jnp.broadcast_to(s[nn], (L,))
            row = jnp.full((L,), qq * N + nn, jnp.int32)
            for dc in range(DC):
                val = plsc.load_gather(rows_v, [row, dc * L + iota])
                accs[dc] = accs[dc] + w_nn * val
        for dc in range(DC):
            plsc.store_scatter(
                aggbuf, [jnp.full((L,), qq, jnp.int32), dc * L + iota],
                accs[dc])
        return carry

    lax.fori_loop(0, QC, q_body, 0)


def _scb_body(e2f, r2f, p_hbm, ent_emb, agg1_out,
              idx_v, p_v, rowsb, r2cb, aggbuf, *sems):
    wid = _wid()
    qb = wid * QPW
    fb = wid * QPW * N
    pltpu.sync_copy(e2f.at[pl.ds(fb, QPW * N)], idx_v)
    pltpu.sync_copy(p_hbm.at[pl.ds(wid * BPW, BPW)], p_v)

    rows = rowsb
    r2c = r2cb

    def issue(c, k):
        dr = pltpu.async_copy(
            ent_emb.at[idx_v.at[pl.ds(c * CHUNK, CHUNK)]], rows[k], sems[k])
        d2 = pltpu.async_copy(r2f.at[pl.ds(fb + c * CHUNK, CHUNK)],
                              r2c[k], sems[k])
        return dr, d2

    def ring_body(cc, carry):
        c0 = NBUF * cc
        descs = [issue(c0 + k, k) for k in range(NBUF)]
        for k in range(NBUF):
            dr, d2 = descs[k]
            dr.wait()
            d2.wait()
            _scb_compute(c0 + k, rows[k], p_v, r2c[k], aggbuf)
            pltpu.sync_copy(aggbuf,
                            agg1_out.at[pl.ds(qb + (c0 + k) * QC, QC)])
        return carry

    lax.fori_loop(0, (QPW * N) // CHUNK // NBUF, ring_body, 0)


def _sc_b(e2f, r2f, P, ent_emb):
    def body(e2f_, r2f_, p_, ent_, out_, idx_v, p_v, *rest):
        rowsb = rest[:NBUF]
        r2cb = rest[NBUF:2 * NBUF]
        aggbuf = rest[2 * NBUF]
        sems = rest[2 * NBUF + 1:]
        _scb_body(e2f_, r2f_, p_, ent_, out_, idx_v, p_v, rowsb, r2cb,
                  aggbuf, *sems)

    return pl.kernel(
        body,
        out_type=jax.ShapeDtypeStruct((B * N, D), jnp.float32),
        mesh=_mesh(),
        compiler_params=pltpu.CompilerParams(needs_layout_passes=False),
        scratch_types=(
            [pltpu.VMEM((QPW * N,), jnp.int32),
             pltpu.VMEM((BPW, NR), jnp.float32)]
            + [pltpu.VMEM((CHUNK, D), jnp.float32) for _ in range(NBUF)]
            + [pltpu.VMEM((CHUNK,), jnp.int32) for _ in range(NBUF)]
            + [pltpu.VMEM((QC, D), jnp.float32)]
            + [pltpu.SemaphoreType.DMA for _ in range(NBUF)]
        ),
    )(e2f, r2f, P, ent_emb)


def _tcp_body(u_ref, rel_ref, p_ref):
    p_ref[...] = lax.dot_general(u_ref[...], rel_ref[...],
                                 (((1,), (1,)), ((), ())),
                                 preferred_element_type=jnp.float32)


def _tc_p(u, rel):
    return pl.pallas_call(
        _tcp_body,
        grid=(1,),
        in_specs=[pl.BlockSpec((B, D), lambda i: (0, 0)),
                  pl.BlockSpec((NR, D), lambda i: (0, 0))],
        out_specs=pl.BlockSpec((B, NR), lambda i: (0, 0)),
        out_shape=jax.ShapeDtypeStruct((B, NR), jnp.float32),
    )(u, rel)


BB = 512


def _softmax(x):
    m = jnp.max(x, axis=-1, keepdims=True)
    e = jnp.exp(x - m)
    return e / jnp.sum(e, axis=-1, keepdims=True)


def _P_of(u, rel):
    return lax.dot_general(u, rel, (((1,), (1,)), ((), ())),
                           preferred_element_type=jnp.float32)


def _tcf_body(u_ref, ev0_ref, ev1_ref, agg1_ref, r1_ref, rel_ref,
              w0_ref, b0_ref, w1_ref, b1_ref, out_ref):
    f32 = jnp.float32
    u = u_ref[...]
    P = _P_of(u, rel_ref[...])
    iota_r = lax.broadcasted_iota(jnp.int32, (1, 1, NR), 2)
    r1 = r1_ref[...]
    oh1 = (r1[:, :, None] == iota_r).astype(f32)
    s0 = _softmax(jnp.sum(oh1 * P[:, None, :], axis=-1))

    ev1 = ev1_ref[...].reshape(BB * N, D)
    w0 = w0_ref[...]
    b0 = b0_ref[...]
    h1 = jax.nn.relu(jnp.dot(ev1 + agg1_ref[...], w0,
                             preferred_element_type=f32) + b0)
    agg0 = jnp.sum(ev1.reshape(BB, N, D) * s0[:, :, None], axis=1)
    h0 = jax.nn.relu(jnp.dot(ev0_ref[...] + agg0, w0,
                             preferred_element_type=f32) + b0)
    agg0b = jnp.sum(h1.reshape(BB, N, D) * s0[:, :, None], axis=1)
    outv = jnp.tanh(jnp.dot(h0 + agg0b, w1_ref[...],
                            preferred_element_type=f32) + b1_ref[...])
    logits = jnp.sum(u * outv, axis=-1)
    out_ref[...] = jax.nn.sigmoid(logits)[None, None, :]


def _tc_final(u, ev0, ev1_3, agg1, r1, rel, W0, b0, W1, b1):
    nb = B // BB
    return pl.pallas_call(
        _tcf_body,
        grid=(nb,),
        in_specs=[
            pl.BlockSpec((BB, D), lambda i: (i, 0)),
            pl.BlockSpec((BB, D), lambda i: (i, 0)),
            pl.BlockSpec((BB, N, D), lambda i: (i, 0, 0)),
            pl.BlockSpec((BB * N, D), lambda i: (i, 0)),
            pl.BlockSpec((BB, N), lambda i: (i, 0)),
            pl.BlockSpec((NR, D), lambda i: (0, 0)),
            pl.BlockSpec((D, D), lambda i: (0, 0)),
            pl.BlockSpec((1, D), lambda i: (0, 0)),
            pl.BlockSpec((D, D), lambda i: (0, 0)),
            pl.BlockSpec((1, D), lambda i: (0, 0)),
        ],
        out_specs=pl.BlockSpec((1, 1, BB), lambda i: (i, 0, 0)),
        out_shape=jax.ShapeDtypeStruct((nb, 1, BB), jnp.float32),
    )(u, ev0, ev1_3, agg1, r1, rel, W0, b0, W1, b1)


def kernel(user_indices, item_indices, adj_entity, adj_relation,
           user_emb, entity_emb, relation_emb, W0, b0, W1, b1):
    adjcat = jnp.concatenate([adj_entity, adj_relation], axis=1)
    adjcat = adjcat.reshape(adj_entity.shape[0] // 4, 128)
    u, ev0, ev1, r1, e2f, r2f = _sc_a(
        user_indices, item_indices, adjcat, user_emb, entity_emb)
    P = _tc_p(u, relation_emb)
    agg1 = _sc_b(e2f, r2f, P, entity_emb)
    out = _tc_final(u, ev0, ev1.reshape(B, N, D), agg1, r1, relation_emb,
                    W0, b0.reshape(1, D), W1, b1.reshape(1, D))
    return out.reshape(B)
